# baseline (speedup 1.0000x reference)
import functools

import jax
import jax.numpy as jnp
from jax import lax
from jax.experimental import pallas as pl
from jax.experimental.pallas import tpu as pltpu
from jax.experimental.pallas import tpu_sc as plsc

_D = 768
_B = 4
_S = 4096
_N = _B * _S
_NC = 2
_NS = 16
_NW = _NC * _NS
_SPAN = _S // _NW
_ROWS_PER_W = _SPAN * _B
_CHUNK = 16
_NCHUNKS = _ROWS_PER_W // _CHUNK
_CHUNKS_PER_B = _SPAN // _CHUNK
_LANES = 16
_VECS_PER_ROW = _D // _LANES


def _make_sc_kernel():
    mesh = plsc.VectorSubcoreMesh(core_axis_name="c", subcore_axis_name="s")

    @functools.partial(
        pl.kernel,
        out_type=jax.ShapeDtypeStruct((_N, _D), jnp.float32),
        mesh=mesh,
        scratch_types=[
            pltpu.VMEM((_ROWS_PER_W,), jnp.int32),
            pltpu.VMEM((_SPAN, _D), jnp.float32),
            pltpu.VMEM((_CHUNK, _D), jnp.float32),
            pltpu.VMEM((_CHUNK, _D), jnp.float32),
            pltpu.SemaphoreType.DMA,
            pltpu.SemaphoreType.DMA,
        ],
    )
    def body(ids_hbm, word_hbm, pos_hbm, out_hbm, idx_v, pos_v,
             rows0, rows1, sg0, sg1):
        wid = lax.axis_index("s") * _NC + lax.axis_index("c")
        s0 = wid * _SPAN
        for b in range(_B):
            pltpu.sync_copy(
                ids_hbm.at[pl.ds(b * _S + s0, _SPAN)],
                idx_v.at[pl.ds(b * _SPAN, _SPAN)],
            )

        def start(c, rows, sg):
            pltpu.async_copy(
                word_hbm.at[idx_v.at[pl.ds(c * _CHUNK, _CHUNK)]], rows, sg)

        start(0, rows0, sg0)
        pltpu.sync_copy(pos_hbm.at[pl.ds(s0, _SPAN)], pos_v)

        def finish(c, rows, sg):
            pltpu.make_async_copy(word_hbm.at[pl.ds(0, _CHUNK)], rows, sg).wait()
            b_idx = c // _CHUNKS_PER_B
            j0 = lax.rem(c, _CHUNKS_PER_B) * _CHUNK

            def row_step(r, carry):
                for k in range(_VECS_PER_ROW):
                    plsc.addupdate(
                        rows.at[r, pl.ds(k * _LANES, _LANES)],
                        pos_v[j0 + r, pl.ds(k * _LANES, _LANES)],
                    )
                return carry

            lax.fori_loop(0, _CHUNK, row_step, 0, unroll=4)
            out_off = b_idx * _S + s0 + j0
            pltpu.sync_copy(rows, out_hbm.at[pl.ds(out_off, _CHUNK)])

        def loop_body(i, carry):
            c0 = i * 2
            start(c0 + 1, rows1, sg1)
            finish(c0, rows0, sg0)

            @pl.when(c0 + 2 < _NCHUNKS)
            def _():
                start(c0 + 2, rows0, sg0)

            finish(c0 + 1, rows1, sg1)
            return carry

        lax.fori_loop(0, _NCHUNKS // 2, loop_body, 0, unroll=False)

    return body


_sc_kernel = _make_sc_kernel()


@jax.jit
def kernel(input_ids, word_embeddings, position_embeddings):
    ids_flat = jnp.reshape(input_ids.astype(jnp.int32), (_N,))
    out = _sc_kernel(ids_flat, word_embeddings, position_embeddings)
    return jnp.reshape(out, (_B, _S, _D))


# speedup vs baseline: 1.6467x; 1.6467x over previous
import functools

import jax
import jax.numpy as jnp
from jax import lax
from jax.experimental import pallas as pl
from jax.experimental.pallas import tpu as pltpu
from jax.experimental.pallas import tpu_sc as plsc

_D = 768
_B = 4
_S = 4096
_N = _B * _S
_NC = 2
_NS = 16
_NW = _NC * _NS
_SPAN = _S // _NW
_CHUNK = 32
_NJB = _SPAN // _CHUNK
_LANES = 16
_VECS_PER_ROW = _D // _LANES


def _make_sc_kernel():
    mesh = plsc.VectorSubcoreMesh(core_axis_name="c", subcore_axis_name="s")

    @functools.partial(
        pl.kernel,
        out_type=jax.ShapeDtypeStruct((_N, _D), jnp.float32),
        mesh=mesh,
        scratch_types=[
            pltpu.VMEM((_B * _SPAN,), jnp.int32),
            pltpu.VMEM((_CHUNK, _D), jnp.float32),
            pltpu.VMEM((_CHUNK, _D), jnp.float32),
            pltpu.VMEM((_CHUNK, _D), jnp.float32),
            pltpu.VMEM((_CHUNK, _D), jnp.float32),
            pltpu.SemaphoreType.DMA,
            pltpu.SemaphoreType.DMA,
            pltpu.SemaphoreType.DMA,
            pltpu.SemaphoreType.DMA,
        ],
    )
    def body(ids_hbm, word_hbm, pos_hbm, out_hbm, idx_v,
             rows0, rows1, pos0, pos1, sg0, sg1, sp0, sp1):
        wid = lax.axis_index("s") * _NC + lax.axis_index("c")
        s0 = wid * _SPAN
        for b in range(_B):
            pltpu.sync_copy(
                ids_hbm.at[pl.ds(b * _S + s0, _SPAN)],
                idx_v.at[pl.ds(b * _SPAN, _SPAN)],
            )

        rows_bufs = (rows0, rows1)
        rows_sems = (sg0, sg1)

        def gstart(b, jb, buf_i):
            pltpu.async_copy(
                word_hbm.at[idx_v.at[pl.ds(b * _SPAN + jb * _CHUNK, _CHUNK)]],
                rows_bufs[buf_i], rows_sems[buf_i])

        def gwait(buf_i):
            pltpu.make_async_copy(
                word_hbm.at[pl.ds(0, _CHUNK)],
                rows_bufs[buf_i], rows_sems[buf_i]).wait()

        def pstart(jb, pos, sp):
            pltpu.async_copy(pos_hbm.at[pl.ds(s0 + jb * _CHUNK, _CHUNK)],
                             pos, sp)

        def pwait(pos, sp):
            pltpu.make_async_copy(pos_hbm.at[pl.ds(0, _CHUNK)], pos, sp).wait()

        def jblock(jb, pos_cur, sp_cur, pos_nxt, sp_nxt, prefetch_pos_cond):
            if prefetch_pos_cond is None:
                pstart(jb + 1, pos_nxt, sp_nxt)
            else:
                @pl.when(prefetch_pos_cond)
                def _():
                    pstart(jb + 1, pos_nxt, sp_nxt)
            pwait(pos_cur, sp_cur)
            for b in range(_B):
                buf_i = b % 2
                if b < _B - 1:
                    gstart(b + 1, jb, 1 - buf_i)
                else:
                    @pl.when(jb + 1 < _NJB)
                    def _():
                        gstart(0, jb + 1, 1 - buf_i)
                gwait(buf_i)
                rows = rows_bufs[buf_i]

                def row_step(r, carry2):
                    for k in range(_VECS_PER_ROW):
                        plsc.addupdate(
                            rows.at[r, pl.ds(k * _LANES, _LANES)],
                            pos_cur[r, pl.ds(k * _LANES, _LANES)],
                        )
                    return carry2

                lax.fori_loop(0, _CHUNK, row_step, 0, unroll=4)

                pltpu.sync_copy(
                    rows, out_hbm.at[pl.ds(b * _S + s0 + jb * _CHUNK, _CHUNK)])

        gstart(0, 0, 0)
        pstart(0, pos0, sp0)

        def loop_body(jj, carry):
            jb = jj * 2
            jblock(jb, pos0, sp0, pos1, sp1, None)
            jblock(jb + 1, pos1, sp1, pos0, sp0, jb + 2 < _NJB)
            return carry

        lax.fori_loop(0, _NJB // 2, loop_body, 0, unroll=False)

    return body


_sc_kernel = _make_sc_kernel()


@jax.jit
def kernel(input_ids, word_embeddings, position_embeddings):
    ids_flat = jnp.reshape(input_ids.astype(jnp.int32), (_N,))
    out = _sc_kernel(ids_flat, word_embeddings, position_embeddings)
    return jnp.reshape(out, (_B, _S, _D))


# speedup vs baseline: 1.6760x; 1.0178x over previous
import functools

import jax
import jax.numpy as jnp
from jax import lax
from jax.experimental import pallas as pl
from jax.experimental.pallas import tpu as pltpu
from jax.experimental.pallas import tpu_sc as plsc

_D = 768
_B = 4
_S = 4096
_N = _B * _S
_NC = 2
_NS = 16
_NW = _NC * _NS
_SPAN = _S // _NW
_CHUNK = 32
_NJB = _SPAN // _CHUNK
_NCH = _NJB * _B
_LANES = 16
_VECS_PER_ROW = _D // _LANES


def _make_sc_kernel():
    mesh = plsc.VectorSubcoreMesh(core_axis_name="c", subcore_axis_name="s")

    @functools.partial(
        pl.kernel,
        out_type=jax.ShapeDtypeStruct((_N, _D), jnp.float32),
        mesh=mesh,
        scratch_types=[
            pltpu.VMEM((_B * _SPAN,), jnp.int32),
            pltpu.VMEM((_CHUNK, _D), jnp.float32),
            pltpu.VMEM((_CHUNK, _D), jnp.float32),
            pltpu.VMEM((_CHUNK, _D), jnp.float32),
            pltpu.VMEM((_CHUNK, _D), jnp.float32),
            pltpu.VMEM((_CHUNK, _D), jnp.float32),
            pltpu.SemaphoreType.DMA,
            pltpu.SemaphoreType.DMA,
            pltpu.SemaphoreType.DMA,
            pltpu.SemaphoreType.DMA,
            pltpu.SemaphoreType.DMA,
            pltpu.SemaphoreType.DMA,
            pltpu.SemaphoreType.DMA,
            pltpu.SemaphoreType.DMA,
        ],
    )
    def body(ids_hbm, word_hbm, pos_hbm, out_hbm, idx_v,
             rows0, rows1, rows2, pos0, pos1,
             sg0, sg1, sg2, st0, st1, st2, sp0, sp1):
        wid = lax.axis_index("s") * _NC + lax.axis_index("c")
        s0 = wid * _SPAN
        rows = (rows0, rows1, rows2)
        sg = (sg0, sg1, sg2)
        st = (st0, st1, st2)
        pos = (pos0, pos1)
        sp = (sp0, sp1)

        for b in range(_B):
            pltpu.sync_copy(
                ids_hbm.at[pl.ds(b * _S + s0, _SPAN)],
                idx_v.at[pl.ds(b * _SPAN, _SPAN)],
            )

        def gstart(c):
            i = c % 3
            b, jb = c % _B, c // _B
            pltpu.async_copy(
                word_hbm.at[idx_v.at[pl.ds(b * _SPAN + jb * _CHUNK, _CHUNK)]],
                rows[i], sg[i])

        def gwait(c):
            i = c % 3
            pltpu.make_async_copy(
                word_hbm.at[pl.ds(0, _CHUNK)], rows[i], sg[i]).wait()

        def stwait(c):
            i = c % 3
            pltpu.make_async_copy(
                rows[i], out_hbm.at[pl.ds(0, _CHUNK)], st[i]).wait()

        def pstart(jb):
            pltpu.async_copy(pos_hbm.at[pl.ds(s0 + jb * _CHUNK, _CHUNK)],
                             pos[jb % 2], sp[jb % 2])

        def pwait(jb):
            pltpu.make_async_copy(pos_hbm.at[pl.ds(0, _CHUNK)],
                                  pos[jb % 2], sp[jb % 2]).wait()

        pstart(0)
        gstart(0)
        gstart(1)

        for c in range(_NCH):
            i = c % 3
            b, jb = c % _B, c // _B
            if b == 0:
                if jb + 1 < _NJB:
                    pstart(jb + 1)
                pwait(jb)
            gwait(c)
            rbuf, pbuf = rows[i], pos[jb % 2]

            def row_step(r, carry, rbuf=rbuf, pbuf=pbuf):
                for k in range(_VECS_PER_ROW):
                    plsc.addupdate(
                        rbuf.at[r, pl.ds(k * _LANES, _LANES)],
                        pbuf[r, pl.ds(k * _LANES, _LANES)],
                    )
                return carry

            lax.fori_loop(0, _CHUNK, row_step, 0, unroll=4)
            pltpu.async_copy(
                rbuf, out_hbm.at[pl.ds(b * _S + s0 + jb * _CHUNK, _CHUNK)],
                st[i])
            if c + 2 < _NCH:
                if c >= 1:
                    stwait(c - 1)
                gstart(c + 2)

        for c in (_NCH - 3, _NCH - 2, _NCH - 1):
            stwait(c)

    return body


_sc_kernel = _make_sc_kernel()


@jax.jit
def kernel(input_ids, word_embeddings, position_embeddings):
    ids_flat = jnp.reshape(input_ids.astype(jnp.int32), (_N,))
    out = _sc_kernel(ids_flat, word_embeddings, position_embeddings)
    return jnp.reshape(out, (_B, _S, _D))


# speedup vs baseline: 1.7036x; 1.0164x over previous
import functools

import jax
import jax.numpy as jnp
from jax import lax
from jax.experimental import pallas as pl
from jax.experimental.pallas import tpu as pltpu
from jax.experimental.pallas import tpu_sc as plsc

_D = 768
_B = 4
_S = 4096
_N = _B * _S
_NC = 2
_NS = 16
_NW = _NC * _NS
_SPAN = _S // _NW
_CHUNK = 32
_NJB = _SPAN // _CHUNK
_NCH = _NJB * _B
_LANES = 16
_VECS_PER_ROW = _D // _LANES


def _make_sc_kernel():
    mesh = plsc.VectorSubcoreMesh(core_axis_name="c", subcore_axis_name="s")

    @functools.partial(
        pl.kernel,
        out_type=jax.ShapeDtypeStruct((_B, _S, _D), jnp.float32),
        mesh=mesh,
        scratch_types=[
            pltpu.VMEM((_B, _SPAN), jnp.int32),
            pltpu.VMEM((_CHUNK, _D), jnp.float32),
            pltpu.VMEM((_CHUNK, _D), jnp.float32),
            pltpu.VMEM((_CHUNK, _D), jnp.float32),
            pltpu.VMEM((_CHUNK, _D), jnp.float32),
            pltpu.VMEM((_CHUNK, _D), jnp.float32),
            pltpu.SemaphoreType.DMA,
            pltpu.SemaphoreType.DMA,
            pltpu.SemaphoreType.DMA,
            pltpu.SemaphoreType.DMA,
            pltpu.SemaphoreType.DMA,
            pltpu.SemaphoreType.DMA,
            pltpu.SemaphoreType.DMA,
            pltpu.SemaphoreType.DMA,
        ],
    )
    def body(ids_hbm, word_hbm, pos_hbm, out_hbm, idx_v,
             rows0, rows1, rows2, pos0, pos1,
             sg0, sg1, sg2, st0, st1, st2, sp0, sp1):
        wid = lax.axis_index("s") * _NC + lax.axis_index("c")
        s0 = wid * _SPAN
        rows = (rows0, rows1, rows2)
        sg = (sg0, sg1, sg2)
        st = (st0, st1, st2)
        pos = (pos0, pos1)
        sp = (sp0, sp1)

        pltpu.sync_copy(ids_hbm.at[:, pl.ds(s0, _SPAN)], idx_v)

        def gstart(c):
            i = c % 3
            b, jb = c % _B, c // _B
            pltpu.async_copy(
                word_hbm.at[idx_v.at[b, pl.ds(jb * _CHUNK, _CHUNK)]],
                rows[i], sg[i])

        def gwait(c):
            i = c % 3
            pltpu.make_async_copy(
                word_hbm.at[pl.ds(0, _CHUNK)], rows[i], sg[i]).wait()

        def stwait(c):
            i = c % 3
            pltpu.make_async_copy(
                rows[i], out_hbm.at[0, pl.ds(0, _CHUNK)], st[i]).wait()

        def pstart(jb):
            pltpu.async_copy(pos_hbm.at[pl.ds(s0 + jb * _CHUNK, _CHUNK)],
                             pos[jb % 2], sp[jb % 2])

        def pwait(jb):
            pltpu.make_async_copy(pos_hbm.at[pl.ds(0, _CHUNK)],
                                  pos[jb % 2], sp[jb % 2]).wait()

        pstart(0)
        gstart(0)
        gstart(1)

        for c in range(_NCH):
            i = c % 3
            b, jb = c % _B, c // _B
            if b == 0:
                if jb + 1 < _NJB:
                    pstart(jb + 1)
                pwait(jb)
            gwait(c)
            rbuf, pbuf = rows[i], pos[jb % 2]

            def row_step(r, carry, rbuf=rbuf, pbuf=pbuf):
                for k in range(_VECS_PER_ROW):
                    plsc.addupdate(
                        rbuf.at[r, pl.ds(k * _LANES, _LANES)],
                        pbuf[r, pl.ds(k * _LANES, _LANES)],
                    )
                return carry

            lax.fori_loop(0, _CHUNK, row_step, 0, unroll=4)
            pltpu.async_copy(
                rbuf, out_hbm.at[b, pl.ds(s0 + jb * _CHUNK, _CHUNK)],
                st[i])
            if c + 2 < _NCH:
                if c >= 1:
                    stwait(c - 1)
                gstart(c + 2)

        for c in (_NCH - 3, _NCH - 2, _NCH - 1):
            stwait(c)

    return body


_sc_kernel = _make_sc_kernel()


@jax.jit
def kernel(input_ids, word_embeddings, position_embeddings):
    return _sc_kernel(input_ids.astype(jnp.int32), word_embeddings,
                      position_embeddings)
